# baseline (speedup 1.0000x reference)
import functools

import jax
import jax.numpy as jnp
from jax import lax
from jax.experimental import pallas as pl
from jax.experimental.pallas import tpu as pltpu
from jax.experimental.pallas import tpu_sc as plsc

_NC = 2
_NS = 16
_NW = _NC * _NS
_LANES = 16

_IMGS = 12
_NBLK = 3 * _IMGS
_SC_BLKS = 2
_ROWS = 8
_UNROLL = 12


@functools.cache
def _get_stage1():
    mesh = plsc.VectorSubcoreMesh(
        core_axis_name="c", subcore_axis_name="s",
        num_cores=_NC, num_subcores=_NS,
    )
    return functools.partial(
        pl.kernel,
        out_type=jax.ShapeDtypeStruct((_NW, _LANES), jnp.float32),
        mesh=mesh,
        scratch_types=[pltpu.VMEM((_ROWS, 384), jnp.float32)] * 5 + [
            pltpu.VMEM((_LANES,), jnp.float32),
            pltpu.SemaphoreType.DMA,
        ],
        compiler_params=pltpu.CompilerParams(
            needs_layout_passes=False, use_tc_tiling_on_sc=True),
    )(_stage1_body)


def _stage1_body(gh_hbm, gah_hbm, pgh_hbm, pga_hbm, msk_hbm, out_hbm,
                 gv, av, qv, bv, mv, resv, sem):
    cid = lax.axis_index("c")
    sid = lax.axis_index("s")
    wid = cid * _NS + sid
    row0 = cid * 128 + sid * _ROWS
    lane = lax.iota(jnp.int32, _LANES)
    zero = jnp.zeros((_LANES,), jnp.float32)

    hs = []
    for src, dst in zip((gh_hbm, gah_hbm, pgh_hbm, pga_hbm, msk_hbm),
                        (gv, av, qv, bv, mv)):
        h = pltpu.make_async_copy(
            src.at[0, pl.ds(row0, _ROWS), :], dst, sem)
        h.start()
        hs.append(h)
    for h in hs:
        h.wait()

    def body(j, carry):
        pg, ng, spg, sng, pa, na, spa, sna = carry
        r = j // 2
        c0 = (j % 2) * (_UNROLL * _LANES)
        for u in range(_UNROLL):
            sl = pl.ds(c0 + u * _LANES, _LANES)
            lg = gv[r, sl]
            la = av[r, sl]
            mk = mv[r, sl]
            dg = qv[r, sl] - lg
            da = bv[r, sl] - la
            plg = dg * dg * mk
            pla = da * da * mk
            pos_g = lg > 0.1
            neg_g = lg < 0.1
            pos_a = la > 0.1
            neg_a = la < 0.1
            pg = pg + jnp.where(pos_g, 1.0, 0.0)
            ng = ng + jnp.where(neg_g, 1.0, 0.0)
            spg = spg + jnp.where(pos_g, plg, 0.0)
            sng = sng + jnp.where(neg_g, plg, 0.0)
            pa = pa + jnp.where(pos_a, 1.0, 0.0)
            na = na + jnp.where(neg_a, 1.0, 0.0)
            spa = spa + jnp.where(pos_a, pla, 0.0)
            sna = sna + jnp.where(neg_a, pla, 0.0)
        return pg, ng, spg, sng, pa, na, spa, sna

    acc = lax.fori_loop(0, _ROWS * 2, body, (zero,) * 8)
    res = zero
    for q in range(8):
        res = jnp.where(lane == q, jnp.sum(acc[q]), res)
    resv[...] = res
    pltpu.sync_copy(resv, out_hbm.at[wid])


def _tc_stats_body(gh_ref, gah_ref, pgh_ref, pga_ref, msk_ref, out_ref):
    lab_g = gh_ref[0]
    lab_a = gah_ref[0]
    mk = msk_ref[0]
    dg = pgh_ref[0] - lab_g
    da = pga_ref[0] - lab_a
    plg = dg * dg * mk
    pla = da * da * mk
    pos_g = lab_g > 0.1
    neg_g = lab_g < 0.1
    pos_a = lab_a > 0.1
    neg_a = lab_a < 0.1
    vals = (
        jnp.sum(pos_g.astype(jnp.float32)),
        jnp.sum(neg_g.astype(jnp.float32)),
        jnp.sum(jnp.where(pos_g, plg, 0.0)),
        jnp.sum(jnp.where(neg_g, plg, 0.0)),
        jnp.sum(pos_a.astype(jnp.float32)),
        jnp.sum(neg_a.astype(jnp.float32)),
        jnp.sum(jnp.where(pos_a, pla, 0.0)),
        jnp.sum(jnp.where(neg_a, pla, 0.0)),
    )
    lane = lax.broadcasted_iota(jnp.int32, (1, 1, 128), 2)
    res = jnp.zeros((1, 1, 128), jnp.float32)
    for q, v in enumerate(vals):
        res = jnp.where(lane == q, v, res)
    out_ref[...] = res


def _tc_stats(gh_label, gah_label, p_gh, p_gah, mask):
    n = _NBLK - _SC_BLKS
    spec = pl.BlockSpec((1, 128, 384),
                        lambda j: ((j + _SC_BLKS) // 3, (j + _SC_BLKS) % 3, 0))
    return pl.pallas_call(
        _tc_stats_body,
        grid=(n,),
        in_specs=[spec] * 5,
        out_specs=pl.BlockSpec((1, 1, 128), lambda j: (j, 0, 0)),
        out_shape=jax.ShapeDtypeStruct((n, 1, 128), jnp.float32),
    )(gh_label, gah_label, p_gh, p_gah, mask)


def _sel_body(k_ref, lab_ref, pred_ref, msk_ref, tk_ref, t5_ref):
    r = pl.program_id(0)
    lab = lab_ref[0]
    d = pred_ref[0] - lab
    plv = d * d * msk_ref[0]
    bits = lax.bitcast_convert_type(plv, jnp.int32)
    negbits = jnp.where(lab < 0.1, bits, -1)
    kk = k_ref[r]

    def srch(b, carry):
        t1, t2 = carry
        bit = jnp.left_shift(jnp.int32(1), 30 - b)
        tr1 = t1 | bit
        tr2 = t2 | bit
        c1 = jnp.sum((negbits >= tr1).astype(jnp.int32))
        c2 = jnp.sum((bits >= tr2).astype(jnp.int32))
        t1 = jnp.where(c1 >= kk, tr1, t1)
        t2 = jnp.where(c2 >= 500, tr2, t2)
        return t1, t2

    t1, t2 = lax.fori_loop(0, 31, srch, (jnp.int32(0), jnp.int32(0)))

    thr1 = lax.bitcast_convert_type(jnp.full((1, 1, 128), t1, jnp.int32),
                                    jnp.float32)
    thr2 = lax.bitcast_convert_type(jnp.full((1, 1, 128), t2, jnp.int32),
                                    jnp.float32)
    gt1 = negbits > t1
    gt2 = bits > t2
    c1 = jnp.sum(gt1.astype(jnp.float32))
    c2 = jnp.sum(gt2.astype(jnp.float32))
    s1 = jnp.sum(jnp.where(gt1, plv, 0.0))
    s2 = jnp.sum(jnp.where(gt2, plv, 0.0))
    tk_ref[...] = s1 + (kk.astype(jnp.float32) - c1) * thr1
    t5_ref[...] = s2 + (500.0 - c2) * thr2


def _selection(kk, labs, preds, mask):
    tk, t5 = pl.pallas_call(
        _sel_body,
        grid=(2 * _IMGS,),
        in_specs=[
            pl.BlockSpec(memory_space=pltpu.SMEM),
            pl.BlockSpec((1, 384, 384), lambda r: (r, 0, 0)),
            pl.BlockSpec((1, 384, 384), lambda r: (r, 0, 0)),
            pl.BlockSpec((1, 384, 384), lambda r: (r % _IMGS, 0, 0)),
        ],
        out_specs=[
            pl.BlockSpec((1, 1, 128), lambda r: (r, 0, 0)),
            pl.BlockSpec((1, 1, 128), lambda r: (r, 0, 0)),
        ],
        out_shape=[
            jax.ShapeDtypeStruct((2 * _IMGS, 1, 128), jnp.float32),
            jax.ShapeDtypeStruct((2 * _IMGS, 1, 128), jnp.float32),
        ],
    )(kk, labs, preds, mask)
    return tk[:, 0, 0], t5[:, 0, 0]


def kernel(gh_label, gah_label, p_gh, p_gah, mask):
    sc_parts = _get_stage1()(gh_label, gah_label, p_gh, p_gah, mask)
    tc_parts = _tc_stats(gh_label, gah_label, p_gh, p_gah, mask)

    sc_tot = sc_parts.reshape(_SC_BLKS, _NS, _LANES).sum(axis=1)[:, :8]
    blks = jnp.concatenate([sc_tot, tc_parts[:, 0, :8]], axis=0)
    tot = blks.reshape(_IMGS, 3, 8).sum(axis=1)

    p = jnp.stack([tot[:, 0], tot[:, 4]])
    n = jnp.stack([tot[:, 1], tot[:, 5]])
    sp = jnp.stack([tot[:, 2], tot[:, 6]])
    sn = jnp.stack([tot[:, 3], tot[:, 7]])
    k = 3.0 * p

    need_sel = jnp.any((p == 0.0) | (n >= k))

    def sel_true(_):
        labs = jnp.concatenate([gh_label, gah_label], axis=0)
        preds = jnp.concatenate([p_gh, p_gah], axis=0)
        kints = k.reshape(-1).astype(jnp.int32)
        return _selection(kints, labs, preds, mask)

    def sel_false(_):
        z = jnp.zeros((2 * _IMGS,), jnp.float32)
        return z, z

    tk, t5 = lax.cond(need_sel, sel_true, sel_false, 0)
    tk = tk.reshape(2, _IMGS)
    t5 = t5.reshape(2, _IMGS)

    posi = sp / p
    nega = jnp.where(n < k, sn / n, tk / k)
    row = jnp.where(p != 0.0, posi + nega, t5 / 500.0)
    return jnp.sum(row) / 12.0


# speedup vs baseline: 1.2498x; 1.2498x over previous
import functools

import jax
import jax.numpy as jnp
from jax import lax
from jax.experimental import pallas as pl
from jax.experimental.pallas import tpu as pltpu
from jax.experimental.pallas import tpu_sc as plsc

_NC = 2
_NS = 16
_NW = _NC * _NS
_LANES = 16

_IMGS = 12
_NBLK = 3 * _IMGS
_SC_BLKS = 2
_ROWS = 8
_UNROLL = 12


@functools.cache
def _get_stage1():
    mesh = plsc.VectorSubcoreMesh(
        core_axis_name="c", subcore_axis_name="s",
        num_cores=_NC, num_subcores=_NS,
    )
    return functools.partial(
        pl.kernel,
        out_type=jax.ShapeDtypeStruct((_NW, _LANES), jnp.float32),
        mesh=mesh,
        scratch_types=[pltpu.VMEM((_ROWS, 384), jnp.float32)] * 5 + [
            pltpu.VMEM((_LANES,), jnp.float32),
            pltpu.SemaphoreType.DMA,
        ],
        compiler_params=pltpu.CompilerParams(
            needs_layout_passes=False, use_tc_tiling_on_sc=True),
    )(_stage1_body)


def _stage1_body(gh_hbm, gah_hbm, pgh_hbm, pga_hbm, msk_hbm, out_hbm,
                 gv, av, qv, bv, mv, resv, sem):
    cid = lax.axis_index("c")
    sid = lax.axis_index("s")
    wid = cid * _NS + sid
    row0 = cid * 128 + sid * _ROWS
    lane = lax.iota(jnp.int32, _LANES)
    zero = jnp.zeros((_LANES,), jnp.float32)

    hs = []
    for src, dst in zip((gh_hbm, gah_hbm, pgh_hbm, pga_hbm, msk_hbm),
                        (gv, av, qv, bv, mv)):
        h = pltpu.make_async_copy(
            src.at[0, pl.ds(row0, _ROWS), :], dst, sem)
        h.start()
        hs.append(h)
    for h in hs:
        h.wait()

    def body(j, carry):
        pg, ng, spg, sng, pa, na, spa, sna = carry
        r = j // 2
        c0 = (j % 2) * (_UNROLL * _LANES)
        for u in range(_UNROLL):
            sl = pl.ds(c0 + u * _LANES, _LANES)
            lg = gv[r, sl]
            la = av[r, sl]
            mk = mv[r, sl]
            dg = qv[r, sl] - lg
            da = bv[r, sl] - la
            plg = dg * dg * mk
            pla = da * da * mk
            pos_g = lg > 0.1
            neg_g = lg < 0.1
            pos_a = la > 0.1
            neg_a = la < 0.1
            pg = pg + jnp.where(pos_g, 1.0, 0.0)
            ng = ng + jnp.where(neg_g, 1.0, 0.0)
            spg = spg + jnp.where(pos_g, plg, 0.0)
            sng = sng + jnp.where(neg_g, plg, 0.0)
            pa = pa + jnp.where(pos_a, 1.0, 0.0)
            na = na + jnp.where(neg_a, 1.0, 0.0)
            spa = spa + jnp.where(pos_a, pla, 0.0)
            sna = sna + jnp.where(neg_a, pla, 0.0)
        return pg, ng, spg, sng, pa, na, spa, sna

    acc = lax.fori_loop(0, _ROWS * 2, body, (zero,) * 8)
    res = zero
    for q in range(8):
        res = jnp.where(lane == q, jnp.sum(acc[q]), res)
    resv[...] = res
    pltpu.sync_copy(resv, out_hbm.at[wid])


def _tc_stats_body(gh_ref, gah_ref, pgh_ref, pga_ref, msk_ref, out_ref):
    lab_g = gh_ref[0]
    lab_a = gah_ref[0]
    mk = msk_ref[0]
    dg = pgh_ref[0] - lab_g
    da = pga_ref[0] - lab_a
    plg = dg * dg * mk
    pla = da * da * mk
    pos_g = lab_g > 0.1
    neg_g = lab_g < 0.1
    pos_a = lab_a > 0.1
    neg_a = lab_a < 0.1
    vals = (
        jnp.sum(pos_g.astype(jnp.float32)),
        jnp.sum(neg_g.astype(jnp.float32)),
        jnp.sum(jnp.where(pos_g, plg, 0.0)),
        jnp.sum(jnp.where(neg_g, plg, 0.0)),
        jnp.sum(pos_a.astype(jnp.float32)),
        jnp.sum(neg_a.astype(jnp.float32)),
        jnp.sum(jnp.where(pos_a, pla, 0.0)),
        jnp.sum(jnp.where(neg_a, pla, 0.0)),
    )
    lane = lax.broadcasted_iota(jnp.int32, (1, 1, 128), 2)
    res = jnp.zeros((1, 1, 128), jnp.float32)
    for q, v in enumerate(vals):
        res = jnp.where(lane == q, v, res)
    out_ref[...] = res


def _tc_stats(gh_label, gah_label, p_gh, p_gah, mask):
    spec = pl.BlockSpec((1, 384, 384), lambda i: (i + 1, 0, 0))
    imgs = pl.pallas_call(
        _tc_stats_body,
        grid=(_IMGS - 1,),
        in_specs=[spec] * 5,
        out_specs=pl.BlockSpec((1, 1, 128), lambda i: (i, 0, 0)),
        out_shape=jax.ShapeDtypeStruct((_IMGS - 1, 1, 128), jnp.float32),
    )(gh_label, gah_label, p_gh, p_gah, mask)
    rspec = pl.BlockSpec((1, 128, 384), lambda i: (0, 2, 0))
    rest = pl.pallas_call(
        _tc_stats_body,
        grid=(1,),
        in_specs=[rspec] * 5,
        out_specs=pl.BlockSpec((1, 1, 128), lambda i: (i, 0, 0)),
        out_shape=jax.ShapeDtypeStruct((1, 1, 128), jnp.float32),
    )(gh_label, gah_label, p_gh, p_gah, mask)
    return imgs, rest


def _sel_body(k_ref, lab_ref, pred_ref, msk_ref, tk_ref, t5_ref):
    r = pl.program_id(0)
    lab = lab_ref[0]
    d = pred_ref[0] - lab
    plv = d * d * msk_ref[0]
    bits = lax.bitcast_convert_type(plv, jnp.int32)
    negbits = jnp.where(lab < 0.1, bits, -1)
    kk = k_ref[r]

    def srch(b, carry):
        t1, t2 = carry
        bit = jnp.left_shift(jnp.int32(1), 30 - b)
        tr1 = t1 | bit
        tr2 = t2 | bit
        c1 = jnp.sum((negbits >= tr1).astype(jnp.int32))
        c2 = jnp.sum((bits >= tr2).astype(jnp.int32))
        t1 = jnp.where(c1 >= kk, tr1, t1)
        t2 = jnp.where(c2 >= 500, tr2, t2)
        return t1, t2

    t1, t2 = lax.fori_loop(0, 31, srch, (jnp.int32(0), jnp.int32(0)))

    thr1 = lax.bitcast_convert_type(jnp.full((1, 1, 128), t1, jnp.int32),
                                    jnp.float32)
    thr2 = lax.bitcast_convert_type(jnp.full((1, 1, 128), t2, jnp.int32),
                                    jnp.float32)
    gt1 = negbits > t1
    gt2 = bits > t2
    c1 = jnp.sum(gt1.astype(jnp.float32))
    c2 = jnp.sum(gt2.astype(jnp.float32))
    s1 = jnp.sum(jnp.where(gt1, plv, 0.0))
    s2 = jnp.sum(jnp.where(gt2, plv, 0.0))
    tk_ref[...] = s1 + (kk.astype(jnp.float32) - c1) * thr1
    t5_ref[...] = s2 + (500.0 - c2) * thr2


def _selection(kk, labs, preds, mask):
    tk, t5 = pl.pallas_call(
        _sel_body,
        grid=(2 * _IMGS,),
        in_specs=[
            pl.BlockSpec(memory_space=pltpu.SMEM),
            pl.BlockSpec((1, 384, 384), lambda r: (r, 0, 0)),
            pl.BlockSpec((1, 384, 384), lambda r: (r, 0, 0)),
            pl.BlockSpec((1, 384, 384), lambda r: (r % _IMGS, 0, 0)),
        ],
        out_specs=[
            pl.BlockSpec((1, 1, 128), lambda r: (r, 0, 0)),
            pl.BlockSpec((1, 1, 128), lambda r: (r, 0, 0)),
        ],
        out_shape=[
            jax.ShapeDtypeStruct((2 * _IMGS, 1, 128), jnp.float32),
            jax.ShapeDtypeStruct((2 * _IMGS, 1, 128), jnp.float32),
        ],
    )(kk, labs, preds, mask)
    return tk[:, 0, 0], t5[:, 0, 0]


def kernel(gh_label, gah_label, p_gh, p_gah, mask):
    sc_parts = _get_stage1()(gh_label, gah_label, p_gh, p_gah, mask)
    tc_imgs, tc_rest = _tc_stats(gh_label, gah_label, p_gh, p_gah, mask)

    img0 = sc_parts.sum(axis=0)[:8] + tc_rest[0, 0, :8]
    tot = jnp.concatenate([img0[None], tc_imgs[:, 0, :8]], axis=0)

    p = jnp.stack([tot[:, 0], tot[:, 4]])
    n = jnp.stack([tot[:, 1], tot[:, 5]])
    sp = jnp.stack([tot[:, 2], tot[:, 6]])
    sn = jnp.stack([tot[:, 3], tot[:, 7]])
    k = 3.0 * p

    need_sel = jnp.any((p == 0.0) | (n >= k))

    def sel_true(_):
        labs = jnp.concatenate([gh_label, gah_label], axis=0)
        preds = jnp.concatenate([p_gh, p_gah], axis=0)
        kints = k.reshape(-1).astype(jnp.int32)
        return _selection(kints, labs, preds, mask)

    def sel_false(_):
        z = jnp.zeros((2 * _IMGS,), jnp.float32)
        return z, z

    tk, t5 = lax.cond(need_sel, sel_true, sel_false, 0)
    tk = tk.reshape(2, _IMGS)
    t5 = t5.reshape(2, _IMGS)

    posi = sp / p
    nega = jnp.where(n < k, sn / n, tk / k)
    row = jnp.where(p != 0.0, posi + nega, t5 / 500.0)
    return jnp.sum(row) / 12.0
